# Initial kernel scaffold; baseline (speedup 1.0000x reference)
import jax
import jax.numpy as jnp
from jax.experimental import pallas as pl


def kernel(H_noisy, X_noisy, cond_embedding, edges, edge_types, generate_mask, batch_ids, beta, params):
    raise NotImplementedError("write your pallas kernel here")


# speedup vs baseline: 1.5094x; 1.5094x over previous
import functools

import numpy as np
import jax
import jax.numpy as jnp
from jax import lax
from jax.experimental import pallas as pl
from jax.experimental.pallas import tpu as pltpu
from jax.experimental.pallas import tpu_sc as plsc

NN = 10000
NP = 10240
EDGES = 160000
EP = 163840
HD = 512
INF = 256
NLAYERS = 3
AVG_DEG = 16.0

BN = 256
BE = 512
NW = 32
GB = 64
NGB = (EP // NW) // GB
SB = 64
SPT = EP // 16
NSB = SPT // SB
NROWS_T = NP // 16

TW = 640
NCH = 5

f32 = jnp.float32


def _inp_body(hn, cond, betab, w1h, w1c, w1ts, w1tc, b1, w2, b2, w3, b3, out):
    half = HD // 2
    k = lax.broadcasted_iota(jnp.int32, (1, half), 1).astype(f32)
    freqs = jnp.exp(-np.log(10000.0) * k / (half - 1))
    args = betab[...] * freqs
    sn = jnp.sin(args)
    cs = jnp.cos(args)
    acc = (jnp.dot(hn[...], w1h[...], preferred_element_type=f32)
           + jnp.dot(cond[...], w1c[...], preferred_element_type=f32)
           + jnp.dot(sn, w1ts[...], preferred_element_type=f32)
           + jnp.dot(cs, w1tc[...], preferred_element_type=f32)
           + b1[...])
    h = jnp.maximum(acc, 0.0)
    h = jnp.maximum(jnp.dot(h, w2[...], preferred_element_type=f32) + b2[...], 0.0)
    out[...] = jnp.dot(h, w3[...], preferred_element_type=f32) + b3[...]


def _full2(shape):
    return pl.BlockSpec(shape, lambda i: (0, 0))


def _input_mlp(hn, cond, betab, w1h, w1c, w1ts, w1tc, b1, w2, b2, w3, b3):
    return pl.pallas_call(
        _inp_body,
        grid=(NP // BN,),
        in_specs=[
            pl.BlockSpec((BN, INF), lambda i: (i, 0)),
            pl.BlockSpec((BN, HD), lambda i: (i, 0)),
            pl.BlockSpec((BN, HD // 2), lambda i: (i, 0)),
            _full2((INF, HD)), _full2((HD, HD)),
            _full2((HD // 2, HD)), _full2((HD // 2, HD)), _full2((1, HD)),
            _full2((HD, HD)), _full2((1, HD)),
            _full2((HD, HD)), _full2((1, HD)),
        ],
        out_specs=pl.BlockSpec((BN, HD), lambda i: (i, 0)),
        out_shape=jax.ShapeDtypeStruct((NP, HD), f32),
    )(hn, cond, betab, w1h, w1c, w1ts, w1tc, b1, w2, b2, w3, b3)


def _nodeA_body(h, x, wd, ws, ad, as_):
    z = jnp.zeros((BN, TW - HD - 16), f32)
    ad[...] = jnp.concatenate(
        [jnp.dot(h[...], wd[...], preferred_element_type=f32), x[...], z], axis=-1)
    as_[...] = jnp.concatenate(
        [jnp.dot(h[...], ws[...], preferred_element_type=f32), x[...], z], axis=-1)


def _node_pre(h, x, wd, ws):
    return pl.pallas_call(
        _nodeA_body,
        grid=(NP // BN,),
        in_specs=[
            pl.BlockSpec((BN, HD), lambda i: (i, 0)),
            pl.BlockSpec((BN, 16), lambda i: (i, 0)),
            _full2((HD, HD)), _full2((HD, HD)),
        ],
        out_specs=[
            pl.BlockSpec((BN, TW), lambda i: (i, 0)),
            pl.BlockSpec((BN, TW), lambda i: (i, 0)),
        ],
        out_shape=[
            jax.ShapeDtypeStruct((NP, TW), f32),
            jax.ShapeDtypeStruct((NP, TW), f32),
        ],
    )(h, x, wd, ws)


def _sc_gather_body(ad_h, as_h, dst_h, src_h,
                    g1_h, g2_h,
                    idxd, idxs, bufd, bufs, sem):
    wid = lax.axis_index("s") * 2 + lax.axis_index("c")
    rbase = wid * NGB
    pltpu.sync_copy(dst_h.at[pl.ds(rbase, NGB)], idxd)
    pltpu.sync_copy(src_h.at[pl.ds(rbase, NGB)], idxs)

    def step(j, carry):
        base = (rbase + j) * GB
        c1 = pltpu.async_copy(ad_h.at[idxd.at[j]], bufd, sem)
        c2 = pltpu.async_copy(as_h.at[idxs.at[j]], bufs, sem)
        c1.wait(); c2.wait()
        pltpu.sync_copy(bufd, g1_h.at[pl.ds(base, GB)])
        pltpu.sync_copy(bufs, g2_h.at[pl.ds(base, GB)])
        return carry

    lax.fori_loop(0, NGB, step, 0)


@functools.lru_cache(maxsize=None)
def _build_sc_gather():
    return pl.kernel(
        _sc_gather_body,
        out_type=(jax.ShapeDtypeStruct((EP, TW), f32),
                  jax.ShapeDtypeStruct((EP, TW), f32)),
        mesh=plsc.VectorSubcoreMesh(core_axis_name="c", subcore_axis_name="s"),
        scratch_types=[
            pltpu.VMEM((NGB, GB), jnp.int32),
            pltpu.VMEM((NGB, GB), jnp.int32),
            pltpu.VMEM((GB, TW), f32),
            pltpu.VMEM((GB, TW), f32),
            pltpu.SemaphoreType.DMA,
        ],
    )


def _sc_gather(ad, as_, dst2, src2):
    return _build_sc_gather()(ad, as_, dst2, src2)


def _edge_body(g1, g2, tyf, et, we, b1, wdist, ew2, eb2, xwt, xbp, mout):
    tvec = jnp.dot(et[...], we[...], preferred_element_type=f32)
    tcol = tyf[...][:, 0:1]
    ga = g1[...]
    gb = g2[...]
    rel = ga[:, HD:HD + 16] - gb[:, HD:HD + 16]
    d2 = jnp.sum(rel * rel, axis=1, keepdims=True)
    sel = tvec[0:1, :] + tcol * (tvec[1:2, :] - tvec[0:1, :])
    d2b = d2.astype(jnp.bfloat16).astype(f32)
    wdb = wdist[...].astype(jnp.bfloat16).astype(f32)
    pre = ga[:, :HD] + gb[:, :HD] + d2b * wdb + sel + b1[...]
    a1 = jnp.maximum(pre, 0.0)
    m = jnp.maximum(jnp.dot(a1, ew2[...], preferred_element_type=f32) + eb2[...], 0.0)
    mb = m.astype(jnp.bfloat16).astype(f32)
    xwb = xwt[...].astype(jnp.bfloat16).astype(f32)
    coef = jnp.sum(mb * xwb, axis=1, keepdims=True) + xbp[0, 0]
    relc = rel * coef
    for c in range(4):
        mout[c, :, :] = m[:, c * 128:(c + 1) * 128]
    mout[4, :, :] = jnp.concatenate([relc, jnp.zeros((BE, 112), f32)], axis=-1)


def _edge_mlp(g1, g2, tyf, et, we, b1, wdist, ew2, eb2, xwt, xbp):
    return pl.pallas_call(
        _edge_body,
        grid=(EP // BE,),
        in_specs=[
            pl.BlockSpec((BE, TW), lambda i: (i, 0)),
            pl.BlockSpec((BE, TW), lambda i: (i, 0)),
            pl.BlockSpec((BE, 16), lambda i: (i, 0)),
            _full2((2, 128)), _full2((128, HD)),
            _full2((1, HD)), _full2((1, HD)),
            _full2((HD, HD)), _full2((1, HD)),
            _full2((1, HD)), _full2((1, 128)),
        ],
        out_specs=pl.BlockSpec((NCH, BE, 128), lambda i: (0, i, 0)),
        out_shape=jax.ShapeDtypeStruct((NCH, EP, 128), f32),
    )(g1, g2, tyf, et, we, b1, wdist, ew2, eb2, xwt, xbp)


def _sc_scatter_body(m5_h, dst_h, z128_h, hm_h, acc, idxb, mbuf):
    c = lax.axis_index("c")
    s = lax.axis_index("s")
    rbase = s * NROWS_T

    def run_chunk(cid):
        pltpu.sync_copy(z128_h.at[pl.ds(rbase, NROWS_T)],
                        acc.at[pl.ds(rbase, NROWS_T)])
        plsc.subcore_barrier()

        def step(j, carry):
            base = s * SPT + j * SB
            pltpu.sync_copy(dst_h.at[s * NSB + j], idxb)
            pltpu.sync_copy(m5_h.at[cid, pl.ds(base, SB)], mbuf)
            pltpu.sync_copy(mbuf, acc.at[idxb], add=True)
            return carry

        lax.fori_loop(0, NSB, step, 0)
        plsc.subcore_barrier()
        pltpu.sync_copy(acc.at[pl.ds(rbase, NROWS_T)],
                        hm_h.at[cid, pl.ds(rbase, NROWS_T)])
        plsc.subcore_barrier()

    for kk in range(2):
        run_chunk(c * 2 + kk)

    @pl.when(c == 0)
    def _():
        run_chunk(4)


@functools.lru_cache(maxsize=None)
def _build_sc_scatter():
    return pl.kernel(
        _sc_scatter_body,
        out_type=jax.ShapeDtypeStruct((NCH, NP, 128), f32),
        mesh=plsc.VectorSubcoreMesh(core_axis_name="c", subcore_axis_name="s"),
        scratch_types=[
            pltpu.VMEM_SHARED((NP, 128), f32),
            pltpu.VMEM((SB,), jnp.int32),
            pltpu.VMEM((SB, 128), f32),
        ],
    )


def _sc_scatter(m5, dst2, z128):
    return _build_sc_scatter()(m5, dst2, z128)


def _nodeE_body(h, hm5, x, w1h, w1m, hb1, w2, hb2, hout, xout):
    hm_a = jnp.concatenate([hm5[0], hm5[1], hm5[2], hm5[3]], axis=-1)
    hu = jnp.maximum(jnp.dot(h[...], w1h[...], preferred_element_type=f32)
                     + jnp.dot(hm_a, w1m[...], preferred_element_type=f32)
                     + hb1[...], 0.0)
    hu = jnp.dot(hu, w2[...], preferred_element_type=f32) + hb2[...]
    hout[...] = h[...] + hu
    xout[...] = x[...] + hm5[4][:, :16] * (1.0 / AVG_DEG)


def _node_update(h, hm5, x, w1h, w1m, hb1, w2, hb2):
    return pl.pallas_call(
        _nodeE_body,
        grid=(NP // BN,),
        in_specs=[
            pl.BlockSpec((BN, HD), lambda i: (i, 0)),
            pl.BlockSpec((NCH, BN, 128), lambda i: (0, i, 0)),
            pl.BlockSpec((BN, 16), lambda i: (i, 0)),
            _full2((HD, HD)), _full2((HD, HD)), _full2((1, HD)),
            _full2((HD, HD)), _full2((1, HD)),
        ],
        out_specs=[
            pl.BlockSpec((BN, HD), lambda i: (i, 0)),
            pl.BlockSpec((BN, 16), lambda i: (i, 0)),
        ],
        out_shape=[
            jax.ShapeDtypeStruct((NP, HD), f32),
            jax.ShapeDtypeStruct((NP, 16), f32),
        ],
    )(h, hm5, x, w1h, w1m, hb1, w2, hb2)


def _head_body(h, x, hn, x0, mh, mx, w, bb, eh, ex):
    nh = jnp.dot(h[...], w[...], preferred_element_type=f32) + bb[...]
    eh[...] = (nh - hn[...]) * mh[...]
    ex[...] = (x[...] - x0[...]) * mx[...]


def _head(h, x, hn, x0, mh, mx, w, bb):
    return pl.pallas_call(
        _head_body,
        grid=(NP // BN,),
        in_specs=[
            pl.BlockSpec((BN, HD), lambda i: (i, 0)),
            pl.BlockSpec((BN, 16), lambda i: (i, 0)),
            pl.BlockSpec((BN, INF), lambda i: (i, 0)),
            pl.BlockSpec((BN, 16), lambda i: (i, 0)),
            pl.BlockSpec((BN, INF), lambda i: (i, 0)),
            pl.BlockSpec((BN, 16), lambda i: (i, 0)),
            _full2((HD, INF)), _full2((1, INF)),
        ],
        out_specs=[
            pl.BlockSpec((BN, INF), lambda i: (i, 0)),
            pl.BlockSpec((BN, 16), lambda i: (i, 0)),
        ],
        out_shape=[
            jax.ShapeDtypeStruct((NP, INF), f32),
            jax.ShapeDtypeStruct((NP, 16), f32),
        ],
    )(h, x, hn, x0, mh, mx, w, bb)


def kernel(H_noisy, X_noisy, cond_embedding, edges, edge_types,
           generate_mask, batch_ids, beta, params):
    p = params
    padN = NP - NN
    hn = jnp.pad(H_noisy, ((0, padN), (0, 0)))
    cond = jnp.pad(cond_embedding, ((0, padN), (0, 0)))
    x0 = jnp.pad(X_noisy, ((0, padN), (0, 13)))
    betab = jnp.broadcast_to(jnp.pad(beta, (0, padN))[:, None], (NP, HD // 2))
    maskf = jnp.pad(generate_mask, (0, padN)).astype(f32)
    mh = jnp.broadcast_to(maskf[:, None], (NP, INF))
    mx = jnp.broadcast_to(maskf[:, None], (NP, 16))

    dst2 = jnp.pad(edges[1], (0, EP - EDGES), constant_values=NN).reshape(EP // 64, 64)
    src2 = jnp.pad(edges[0], (0, EP - EDGES), constant_values=NN).reshape(EP // 64, 64)
    tyf = jnp.broadcast_to(
        jnp.pad(edge_types, (0, EP - EDGES)).astype(f32)[:, None], (EP, 16))
    z128 = jnp.zeros((NP, 128), f32)

    w1 = p['inp_w1']
    h = _input_mlp(hn, cond, betab,
                   w1[0:INF], w1[INF:INF + HD],
                   w1[INF + HD:INF + HD + HD // 2], w1[INF + HD + HD // 2:],
                   p['inp_b1'].reshape(1, HD),
                   p['inp_w2'], p['inp_b2'].reshape(1, HD),
                   p['inp_w3'], p['inp_b3'].reshape(1, HD))
    x = x0

    for l in range(NLAYERS):
        ew1 = p[f'l{l}_ew1']
        wd = ew1[0:HD]
        ws = ew1[HD:2 * HD]
        wdist = ew1[2 * HD:2 * HD + 1].reshape(1, HD)
        we = ew1[2 * HD + 1:]
        xbp = jnp.pad(p[f'l{l}_xb'].reshape(1, 1), ((0, 0), (0, 127)))

        ad, as_ = _node_pre(h, x, wd, ws)
        g1, g2 = _sc_gather(ad, as_, dst2, src2)
        m5 = _edge_mlp(g1, g2, tyf,
                       p['edge_table'], we,
                       p[f'l{l}_eb1'].reshape(1, HD), wdist,
                       p[f'l{l}_ew2'], p[f'l{l}_eb2'].reshape(1, HD),
                       p[f'l{l}_xw'].reshape(1, HD), xbp)
        hm5 = _sc_scatter(m5, dst2, z128)
        hw1 = p[f'l{l}_hw1']
        h, x = _node_update(h, hm5, x,
                            hw1[0:HD], hw1[HD:],
                            p[f'l{l}_hb1'].reshape(1, HD),
                            p[f'l{l}_hw2'], p[f'l{l}_hb2'].reshape(1, HD))

    eh, ex = _head(h, x, hn, x0, mh, mx,
                   p['h2i_w'], p['h2i_b'].reshape(1, INF))
    return eh[:NN], ex[:NN, :3]


# speedup vs baseline: 2.0007x; 1.3254x over previous
import functools

import numpy as np
import jax
import jax.numpy as jnp
from jax import lax
from jax.experimental import pallas as pl
from jax.experimental.pallas import tpu as pltpu
from jax.experimental.pallas import tpu_sc as plsc

NN = 10000
NP = 10240
EDGES = 160000
EP = 163840
HD = 512
INF = 256
NLAYERS = 3
AVG_DEG = 16.0

BN = 256
BE = 512
NW = 32
GB = 32
NGB = (EP // NW) // GB
SB = 128
SPT = EP // 16
NSB = SPT // SB
NROWS_T = NP // 16

TW = 640
NCH = 5

f32 = jnp.float32


def _inp_body(hn, cond, betab, w1h, w1c, w1ts, w1tc, b1, w2, b2, w3, b3, out):
    half = HD // 2
    k = lax.broadcasted_iota(jnp.int32, (1, half), 1).astype(f32)
    freqs = jnp.exp(-np.log(10000.0) * k / (half - 1))
    args = betab[...] * freqs
    sn = jnp.sin(args)
    cs = jnp.cos(args)
    acc = (jnp.dot(hn[...], w1h[...], preferred_element_type=f32)
           + jnp.dot(cond[...], w1c[...], preferred_element_type=f32)
           + jnp.dot(sn, w1ts[...], preferred_element_type=f32)
           + jnp.dot(cs, w1tc[...], preferred_element_type=f32)
           + b1[...])
    h = jnp.maximum(acc, 0.0)
    h = jnp.maximum(jnp.dot(h, w2[...], preferred_element_type=f32) + b2[...], 0.0)
    out[...] = jnp.dot(h, w3[...], preferred_element_type=f32) + b3[...]


def _full2(shape):
    return pl.BlockSpec(shape, lambda i: (0, 0))


def _input_mlp(hn, cond, betab, w1h, w1c, w1ts, w1tc, b1, w2, b2, w3, b3):
    return pl.pallas_call(
        _inp_body,
        grid=(NP // BN,),
        in_specs=[
            pl.BlockSpec((BN, INF), lambda i: (i, 0)),
            pl.BlockSpec((BN, HD), lambda i: (i, 0)),
            pl.BlockSpec((BN, HD // 2), lambda i: (i, 0)),
            _full2((INF, HD)), _full2((HD, HD)),
            _full2((HD // 2, HD)), _full2((HD // 2, HD)), _full2((1, HD)),
            _full2((HD, HD)), _full2((1, HD)),
            _full2((HD, HD)), _full2((1, HD)),
        ],
        out_specs=pl.BlockSpec((BN, HD), lambda i: (i, 0)),
        out_shape=jax.ShapeDtypeStruct((NP, HD), f32),
    )(hn, cond, betab, w1h, w1c, w1ts, w1tc, b1, w2, b2, w3, b3)


def _nodeA_body(h, x, wd, ws, ad, as_):
    z = jnp.zeros((BN, TW - HD - 16), f32)
    ad[...] = jnp.concatenate(
        [jnp.dot(h[...], wd[...], preferred_element_type=f32), x[...], z], axis=-1)
    as_[...] = jnp.concatenate(
        [jnp.dot(h[...], ws[...], preferred_element_type=f32), x[...], z], axis=-1)


def _node_pre(h, x, wd, ws):
    return pl.pallas_call(
        _nodeA_body,
        grid=(NP // BN,),
        in_specs=[
            pl.BlockSpec((BN, HD), lambda i: (i, 0)),
            pl.BlockSpec((BN, 16), lambda i: (i, 0)),
            _full2((HD, HD)), _full2((HD, HD)),
        ],
        out_specs=[
            pl.BlockSpec((BN, TW), lambda i: (i, 0)),
            pl.BlockSpec((BN, TW), lambda i: (i, 0)),
        ],
        out_shape=[
            jax.ShapeDtypeStruct((NP, TW), f32),
            jax.ShapeDtypeStruct((NP, TW), f32),
        ],
    )(h, x, wd, ws)


def _sc_gather_body(ad_h, as_h, dst_h, src_h, g1_h, g2_h,
                    idxd, idxs, bufd0, bufs0, bufd1, bufs1,
                    gsem0, gsem1, wsem0, wsem1):
    wid = lax.axis_index("s") * 2 + lax.axis_index("c")
    rbase = wid * NGB
    pltpu.sync_copy(dst_h.at[pl.ds(rbase, NGB)], idxd)
    pltpu.sync_copy(src_h.at[pl.ds(rbase, NGB)], idxs)

    slots = ((bufd0, bufs0, gsem0, wsem0), (bufd1, bufs1, gsem1, wsem1))

    def fire_gather(j, b):
        bd, bs, gs, _ = slots[b]
        pltpu.async_copy(ad_h.at[idxd.at[j]], bd, gs)
        pltpu.async_copy(as_h.at[idxs.at[j]], bs, gs)

    def wait_gather(b):
        bd, bs, gs, _ = slots[b]
        pltpu.make_async_copy(ad_h.at[idxd.at[0]], bd, gs).wait()
        pltpu.make_async_copy(as_h.at[idxs.at[0]], bs, gs).wait()

    def fire_wb(j, b):
        bd, bs, _, ws = slots[b]
        base = (rbase + j) * GB
        pltpu.async_copy(bd, g1_h.at[pl.ds(base, GB)], ws)
        pltpu.async_copy(bs, g2_h.at[pl.ds(base, GB)], ws)

    def wait_wb(b):
        bd, bs, _, ws = slots[b]
        pltpu.make_async_copy(bd, g1_h.at[pl.ds(0, GB)], ws).wait()
        pltpu.make_async_copy(bs, g2_h.at[pl.ds(0, GB)], ws).wait()

    fire_gather(0, 0)

    def body(jj, carry):
        for b in range(2):
            j = jj * 2 + b
            wait_gather(b)
            fire_wb(j, b)
            if b == 0:
                @pl.when(jj > 0)
                def _():
                    wait_wb(1)
            else:
                wait_wb(0)

            @pl.when(j + 1 < NGB)
            def _():
                fire_gather(j + 1, 1 - b)
        return carry

    lax.fori_loop(0, NGB // 2, body, 0)
    wait_wb(1)


@functools.lru_cache(maxsize=None)
def _build_sc_gather():
    return pl.kernel(
        _sc_gather_body,
        out_type=(jax.ShapeDtypeStruct((EP, TW), f32),
                  jax.ShapeDtypeStruct((EP, TW), f32)),
        mesh=plsc.VectorSubcoreMesh(core_axis_name="c", subcore_axis_name="s"),
        scratch_types=[
            pltpu.VMEM((NGB, GB), jnp.int32),
            pltpu.VMEM((NGB, GB), jnp.int32),
            pltpu.VMEM((GB, TW), f32),
            pltpu.VMEM((GB, TW), f32),
            pltpu.VMEM((GB, TW), f32),
            pltpu.VMEM((GB, TW), f32),
            pltpu.SemaphoreType.DMA,
            pltpu.SemaphoreType.DMA,
            pltpu.SemaphoreType.DMA,
            pltpu.SemaphoreType.DMA,
        ],
    )


def _sc_gather(ad, as_, dst2, src2):
    return _build_sc_gather()(ad, as_, dst2, src2)


def _edge_body(g1, g2, tyf, et, we, b1, wdist, ew2, eb2, xwt, xbp, mout):
    tvec = jnp.dot(et[...], we[...], preferred_element_type=f32)
    tcol = tyf[...][:, 0:1]
    ga = g1[...]
    gb = g2[...]
    rel = ga[:, HD:HD + 16] - gb[:, HD:HD + 16]
    d2 = jnp.sum(rel * rel, axis=1, keepdims=True)
    sel = tvec[0:1, :] + tcol * (tvec[1:2, :] - tvec[0:1, :])
    d2b = d2.astype(jnp.bfloat16).astype(f32)
    wdb = wdist[...].astype(jnp.bfloat16).astype(f32)
    pre = ga[:, :HD] + gb[:, :HD] + d2b * wdb + sel + b1[...]
    a1 = jnp.maximum(pre, 0.0)
    m = jnp.maximum(jnp.dot(a1, ew2[...], preferred_element_type=f32) + eb2[...], 0.0)
    mb = m.astype(jnp.bfloat16).astype(f32)
    xwb = xwt[...].astype(jnp.bfloat16).astype(f32)
    coef = jnp.sum(mb * xwb, axis=1, keepdims=True) + xbp[0, 0]
    relc = rel * coef
    for c in range(4):
        mout[c, :, :] = m[:, c * 128:(c + 1) * 128]
    mout[4, :, :] = jnp.concatenate([relc, jnp.zeros((BE, 112), f32)], axis=-1)


def _edge_mlp(g1, g2, tyf, et, we, b1, wdist, ew2, eb2, xwt, xbp):
    return pl.pallas_call(
        _edge_body,
        grid=(EP // BE,),
        in_specs=[
            pl.BlockSpec((BE, TW), lambda i: (i, 0)),
            pl.BlockSpec((BE, TW), lambda i: (i, 0)),
            pl.BlockSpec((BE, 16), lambda i: (i, 0)),
            _full2((2, 128)), _full2((128, HD)),
            _full2((1, HD)), _full2((1, HD)),
            _full2((HD, HD)), _full2((1, HD)),
            _full2((1, HD)), _full2((1, 128)),
        ],
        out_specs=pl.BlockSpec((NCH, BE, 128), lambda i: (0, i, 0)),
        out_shape=jax.ShapeDtypeStruct((NCH, EP, 128), f32),
    )(g1, g2, tyf, et, we, b1, wdist, ew2, eb2, xwt, xbp)


def _sc_scatter_body(m5_h, dst_h, z128_h, hm_h, acc,
                     idx0, idx1, mb0, mb1,
                     lsem0, lsem1, ssem0, ssem1):
    c = lax.axis_index("c")
    s = lax.axis_index("s")
    rbase = s * NROWS_T

    slots = ((idx0, mb0, lsem0, ssem0), (idx1, mb1, lsem1, ssem1))

    def fire_load(cid, j, b):
        ib, mb, ls, _ = slots[b]
        pltpu.async_copy(dst_h.at[s * NSB + j], ib, ls)
        pltpu.async_copy(m5_h.at[cid, pl.ds(s * SPT + j * SB, SB)], mb, ls)

    def wait_load(b):
        ib, mb, ls, _ = slots[b]
        pltpu.make_async_copy(dst_h.at[0], ib, ls).wait()
        pltpu.make_async_copy(m5_h.at[0, pl.ds(0, SB)], mb, ls).wait()

    def fire_sadd(b):
        ib, mb, _, ss = slots[b]
        pltpu.async_copy(mb, acc.at[ib], ss, add=True)

    def wait_sadd(b):
        ib, mb, _, ss = slots[b]
        pltpu.make_async_copy(mb, acc.at[ib], ss).wait()

    def run_chunk(cid):
        pltpu.sync_copy(z128_h.at[pl.ds(rbase, NROWS_T)],
                        acc.at[pl.ds(rbase, NROWS_T)])
        plsc.subcore_barrier()
        fire_load(cid, 0, 0)

        def body(jj, carry):
            for b in range(2):
                j = jj * 2 + b
                wait_load(b)
                fire_sadd(b)
                if b == 0:
                    @pl.when(jj > 0)
                    def _():
                        wait_sadd(1)
                else:
                    wait_sadd(0)

                @pl.when(j + 1 < NSB)
                def _():
                    fire_load(cid, j + 1, 1 - b)
            return carry

        lax.fori_loop(0, NSB // 2, body, 0)
        wait_sadd(1)
        plsc.subcore_barrier()
        pltpu.sync_copy(acc.at[pl.ds(rbase, NROWS_T)],
                        hm_h.at[cid, pl.ds(rbase, NROWS_T)])
        plsc.subcore_barrier()

    for kk in range(2):
        run_chunk(c * 2 + kk)

    @pl.when(c == 0)
    def _():
        run_chunk(4)


@functools.lru_cache(maxsize=None)
def _build_sc_scatter():
    return pl.kernel(
        _sc_scatter_body,
        out_type=jax.ShapeDtypeStruct((NCH, NP, 128), f32),
        mesh=plsc.VectorSubcoreMesh(core_axis_name="c", subcore_axis_name="s"),
        scratch_types=[
            pltpu.VMEM_SHARED((NP, 128), f32),
            pltpu.VMEM((SB,), jnp.int32),
            pltpu.VMEM((SB,), jnp.int32),
            pltpu.VMEM((SB, 128), f32),
            pltpu.VMEM((SB, 128), f32),
            pltpu.SemaphoreType.DMA,
            pltpu.SemaphoreType.DMA,
            pltpu.SemaphoreType.DMA,
            pltpu.SemaphoreType.DMA,
        ],
    )


def _sc_scatter(m5, dst2, z128):
    return _build_sc_scatter()(m5, dst2, z128)


def _nodeE_body(h, hm5, x, w1h, w1m, hb1, w2, hb2, hout, xout):
    hm_a = jnp.concatenate([hm5[0], hm5[1], hm5[2], hm5[3]], axis=-1)
    hu = jnp.maximum(jnp.dot(h[...], w1h[...], preferred_element_type=f32)
                     + jnp.dot(hm_a, w1m[...], preferred_element_type=f32)
                     + hb1[...], 0.0)
    hu = jnp.dot(hu, w2[...], preferred_element_type=f32) + hb2[...]
    hout[...] = h[...] + hu
    xout[...] = x[...] + hm5[4][:, :16] * (1.0 / AVG_DEG)


def _node_update(h, hm5, x, w1h, w1m, hb1, w2, hb2):
    return pl.pallas_call(
        _nodeE_body,
        grid=(NP // BN,),
        in_specs=[
            pl.BlockSpec((BN, HD), lambda i: (i, 0)),
            pl.BlockSpec((NCH, BN, 128), lambda i: (0, i, 0)),
            pl.BlockSpec((BN, 16), lambda i: (i, 0)),
            _full2((HD, HD)), _full2((HD, HD)), _full2((1, HD)),
            _full2((HD, HD)), _full2((1, HD)),
        ],
        out_specs=[
            pl.BlockSpec((BN, HD), lambda i: (i, 0)),
            pl.BlockSpec((BN, 16), lambda i: (i, 0)),
        ],
        out_shape=[
            jax.ShapeDtypeStruct((NP, HD), f32),
            jax.ShapeDtypeStruct((NP, 16), f32),
        ],
    )(h, hm5, x, w1h, w1m, hb1, w2, hb2)


def _head_body(h, x, hn, x0, mh, mx, w, bb, eh, ex):
    nh = jnp.dot(h[...], w[...], preferred_element_type=f32) + bb[...]
    eh[...] = (nh - hn[...]) * mh[...]
    ex[...] = (x[...] - x0[...]) * mx[...]


def _head(h, x, hn, x0, mh, mx, w, bb):
    return pl.pallas_call(
        _head_body,
        grid=(NP // BN,),
        in_specs=[
            pl.BlockSpec((BN, HD), lambda i: (i, 0)),
            pl.BlockSpec((BN, 16), lambda i: (i, 0)),
            pl.BlockSpec((BN, INF), lambda i: (i, 0)),
            pl.BlockSpec((BN, 16), lambda i: (i, 0)),
            pl.BlockSpec((BN, INF), lambda i: (i, 0)),
            pl.BlockSpec((BN, 16), lambda i: (i, 0)),
            _full2((HD, INF)), _full2((1, INF)),
        ],
        out_specs=[
            pl.BlockSpec((BN, INF), lambda i: (i, 0)),
            pl.BlockSpec((BN, 16), lambda i: (i, 0)),
        ],
        out_shape=[
            jax.ShapeDtypeStruct((NP, INF), f32),
            jax.ShapeDtypeStruct((NP, 16), f32),
        ],
    )(h, x, hn, x0, mh, mx, w, bb)


def kernel(H_noisy, X_noisy, cond_embedding, edges, edge_types,
           generate_mask, batch_ids, beta, params):
    p = params
    padN = NP - NN
    hn = jnp.pad(H_noisy, ((0, padN), (0, 0)))
    cond = jnp.pad(cond_embedding, ((0, padN), (0, 0)))
    x0 = jnp.pad(X_noisy, ((0, padN), (0, 13)))
    betab = jnp.broadcast_to(jnp.pad(beta, (0, padN))[:, None], (NP, HD // 2))
    maskf = jnp.pad(generate_mask, (0, padN)).astype(f32)
    mh = jnp.broadcast_to(maskf[:, None], (NP, INF))
    mx = jnp.broadcast_to(maskf[:, None], (NP, 16))

    dstp = jnp.pad(edges[1], (0, EP - EDGES), constant_values=NN)
    srcp = jnp.pad(edges[0], (0, EP - EDGES), constant_values=NN)
    dstg = dstp.reshape(EP // GB, GB)
    srcg = srcp.reshape(EP // GB, GB)
    dsts = dstp.reshape(EP // SB, SB)
    tyf = jnp.broadcast_to(
        jnp.pad(edge_types, (0, EP - EDGES)).astype(f32)[:, None], (EP, 16))
    z128 = jnp.zeros((NP, 128), f32)

    w1 = p['inp_w1']
    h = _input_mlp(hn, cond, betab,
                   w1[0:INF], w1[INF:INF + HD],
                   w1[INF + HD:INF + HD + HD // 2], w1[INF + HD + HD // 2:],
                   p['inp_b1'].reshape(1, HD),
                   p['inp_w2'], p['inp_b2'].reshape(1, HD),
                   p['inp_w3'], p['inp_b3'].reshape(1, HD))
    x = x0

    for l in range(NLAYERS):
        ew1 = p[f'l{l}_ew1']
        wd = ew1[0:HD]
        ws = ew1[HD:2 * HD]
        wdist = ew1[2 * HD:2 * HD + 1].reshape(1, HD)
        we = ew1[2 * HD + 1:]
        xbp = jnp.pad(p[f'l{l}_xb'].reshape(1, 1), ((0, 0), (0, 127)))

        ad, as_ = _node_pre(h, x, wd, ws)
        g1, g2 = _sc_gather(ad, as_, dstg, srcg)
        m5 = _edge_mlp(g1, g2, tyf,
                       p['edge_table'], we,
                       p[f'l{l}_eb1'].reshape(1, HD), wdist,
                       p[f'l{l}_ew2'], p[f'l{l}_eb2'].reshape(1, HD),
                       p[f'l{l}_xw'].reshape(1, HD), xbp)
        hm5 = _sc_scatter(m5, dsts, z128)
        hw1 = p[f'l{l}_hw1']
        h, x = _node_update(h, hm5, x,
                            hw1[0:HD], hw1[HD:],
                            p[f'l{l}_hb1'].reshape(1, HD),
                            p[f'l{l}_hw2'], p[f'l{l}_hb2'].reshape(1, HD))

    eh, ex = _head(h, x, hn, x0, mh, mx,
                   p['h2i_w'], p['h2i_b'].reshape(1, INF))
    return eh[:NN], ex[:NN, :3]


# speedup vs baseline: 2.2667x; 1.1330x over previous
import functools

import numpy as np
import jax
import jax.numpy as jnp
from jax import lax
from jax.experimental import pallas as pl
from jax.experimental.pallas import tpu as pltpu
from jax.experimental.pallas import tpu_sc as plsc

NN = 10000
NP = 10240
EDGES = 160000
EP = 163840
HD = 512
INF = 256
NLAYERS = 3
AVG_DEG = 16.0

BN = 256
BE = 512
NW = 32
GB = 32
NGB = (EP // NW) // GB
SB = 128
SPT = EP // 16
NSB = SPT // SB
NROWS_T = NP // 16

TW = 640
NCH = 5

f32 = jnp.float32


def _inp_body(hn, cond, betab, w1h, w1c, w1ts, w1tc, b1, w2, b2, w3, b3, out):
    half = HD // 2
    k = lax.broadcasted_iota(jnp.int32, (1, half), 1).astype(f32)
    freqs = jnp.exp(-np.log(10000.0) * k / (half - 1))
    args = betab[...] * freqs
    sn = jnp.sin(args)
    cs = jnp.cos(args)
    acc = (jnp.dot(hn[...], w1h[...], preferred_element_type=f32)
           + jnp.dot(cond[...], w1c[...], preferred_element_type=f32)
           + jnp.dot(sn, w1ts[...], preferred_element_type=f32)
           + jnp.dot(cs, w1tc[...], preferred_element_type=f32)
           + b1[...])
    h = jnp.maximum(acc, 0.0)
    h = jnp.maximum(jnp.dot(h, w2[...], preferred_element_type=f32) + b2[...], 0.0)
    out[...] = jnp.dot(h, w3[...], preferred_element_type=f32) + b3[...]


def _full2(shape):
    return pl.BlockSpec(shape, lambda i: (0, 0))


def _input_mlp(hn, cond, betab, w1h, w1c, w1ts, w1tc, b1, w2, b2, w3, b3):
    return pl.pallas_call(
        _inp_body,
        grid=(NP // BN,),
        in_specs=[
            pl.BlockSpec((BN, INF), lambda i: (i, 0)),
            pl.BlockSpec((BN, HD), lambda i: (i, 0)),
            pl.BlockSpec((BN, HD // 2), lambda i: (i, 0)),
            _full2((INF, HD)), _full2((HD, HD)),
            _full2((HD // 2, HD)), _full2((HD // 2, HD)), _full2((1, HD)),
            _full2((HD, HD)), _full2((1, HD)),
            _full2((HD, HD)), _full2((1, HD)),
        ],
        out_specs=pl.BlockSpec((BN, HD), lambda i: (i, 0)),
        out_shape=jax.ShapeDtypeStruct((NP, HD), f32),
    )(hn, cond, betab, w1h, w1c, w1ts, w1tc, b1, w2, b2, w3, b3)


def _nodeA_body(h, x, wd, ws, ad, as_):
    z = jnp.zeros((BN, TW - HD - 16), f32)
    ad[...] = jnp.concatenate(
        [jnp.dot(h[...], wd[...], preferred_element_type=f32), x[...], z], axis=-1)
    as_[...] = jnp.concatenate(
        [jnp.dot(h[...], ws[...], preferred_element_type=f32), x[...], z], axis=-1)


def _node_pre(h, x, wd, ws):
    return pl.pallas_call(
        _nodeA_body,
        grid=(NP // BN,),
        in_specs=[
            pl.BlockSpec((BN, HD), lambda i: (i, 0)),
            pl.BlockSpec((BN, 16), lambda i: (i, 0)),
            _full2((HD, HD)), _full2((HD, HD)),
        ],
        out_specs=[
            pl.BlockSpec((BN, TW), lambda i: (i, 0)),
            pl.BlockSpec((BN, TW), lambda i: (i, 0)),
        ],
        out_shape=[
            jax.ShapeDtypeStruct((NP, TW), f32),
            jax.ShapeDtypeStruct((NP, TW), f32),
        ],
    )(h, x, wd, ws)


def _sc_gather_body(ngb, ad_h, as_h, dst_h, src_h, g1_h, g2_h,
                    idxd, idxs, bufd0, bufs0, bufd1, bufs1,
                    gsem0, gsem1, wsem0, wsem1):
    wid = lax.axis_index("s") * 2 + lax.axis_index("c")
    rbase = wid * ngb
    pltpu.sync_copy(dst_h.at[pl.ds(rbase, ngb)], idxd)
    pltpu.sync_copy(src_h.at[pl.ds(rbase, ngb)], idxs)

    slots = ((bufd0, bufs0, gsem0, wsem0), (bufd1, bufs1, gsem1, wsem1))

    def fire_gather(j, b):
        bd, bs, gs, _ = slots[b]
        pltpu.async_copy(ad_h.at[idxd.at[j]], bd, gs)
        pltpu.async_copy(as_h.at[idxs.at[j]], bs, gs)

    def wait_gather(b):
        bd, bs, gs, _ = slots[b]
        pltpu.make_async_copy(ad_h.at[idxd.at[0]], bd, gs).wait()
        pltpu.make_async_copy(as_h.at[idxs.at[0]], bs, gs).wait()

    def fire_wb(j, b):
        bd, bs, _, ws = slots[b]
        base = (rbase + j) * GB
        pltpu.async_copy(bd, g1_h.at[pl.ds(base, GB)], ws)
        pltpu.async_copy(bs, g2_h.at[pl.ds(base, GB)], ws)

    def wait_wb(b):
        bd, bs, _, ws = slots[b]
        pltpu.make_async_copy(bd, g1_h.at[pl.ds(0, GB)], ws).wait()
        pltpu.make_async_copy(bs, g2_h.at[pl.ds(0, GB)], ws).wait()

    fire_gather(0, 0)

    def body(jj, carry):
        for b in range(2):
            j = jj * 2 + b
            wait_gather(b)
            fire_wb(j, b)
            if b == 0:
                @pl.when(jj > 0)
                def _():
                    wait_wb(1)
            else:
                wait_wb(0)

            @pl.when(j + 1 < ngb)
            def _():
                fire_gather(j + 1, 1 - b)
        return carry

    lax.fori_loop(0, ngb // 2, body, 0)
    wait_wb(1)


@functools.lru_cache(maxsize=None)
def _build_sc_gather(ne):
    ngb = (ne // NW) // GB
    return pl.kernel(
        functools.partial(_sc_gather_body, ngb),
        out_type=(jax.ShapeDtypeStruct((ne, TW), f32),
                  jax.ShapeDtypeStruct((ne, TW), f32)),
        mesh=plsc.VectorSubcoreMesh(core_axis_name="c", subcore_axis_name="s"),
        scratch_types=[
            pltpu.VMEM((ngb, GB), jnp.int32),
            pltpu.VMEM((ngb, GB), jnp.int32),
            pltpu.VMEM((GB, TW), f32),
            pltpu.VMEM((GB, TW), f32),
            pltpu.VMEM((GB, TW), f32),
            pltpu.VMEM((GB, TW), f32),
            pltpu.SemaphoreType.DMA,
            pltpu.SemaphoreType.DMA,
            pltpu.SemaphoreType.DMA,
            pltpu.SemaphoreType.DMA,
        ],
    )


def _sc_gather(ad, as_, dst2, src2):
    return _build_sc_gather(dst2.shape[0] * GB)(ad, as_, dst2, src2)


def _edge_body(g1, g2, tyf, et, we, b1, wdist, ew2, eb2, xwt, xbp, mout):
    tvec = jnp.dot(et[...], we[...], preferred_element_type=f32)
    tcol = tyf[...][:, 0:1]
    ga = g1[...]
    gb = g2[...]
    rel = ga[:, HD:HD + 16] - gb[:, HD:HD + 16]
    d2 = jnp.sum(rel * rel, axis=1, keepdims=True)
    sel = tvec[0:1, :] + tcol * (tvec[1:2, :] - tvec[0:1, :])
    d2b = d2.astype(jnp.bfloat16).astype(f32)
    wdb = wdist[...].astype(jnp.bfloat16).astype(f32)
    pre = ga[:, :HD] + gb[:, :HD] + d2b * wdb + sel + b1[...]
    a1 = jnp.maximum(pre, 0.0)
    m = jnp.maximum(jnp.dot(a1, ew2[...], preferred_element_type=f32) + eb2[...], 0.0)
    mb = m.astype(jnp.bfloat16).astype(f32)
    xwb = xwt[...].astype(jnp.bfloat16).astype(f32)
    coef = jnp.sum(mb * xwb, axis=1, keepdims=True) + xbp[0, 0]
    relc = rel * coef
    for c in range(4):
        mout[c, :, :] = m[:, c * 128:(c + 1) * 128]
    mout[4, :, :] = jnp.concatenate([relc, jnp.zeros((BE, 112), f32)], axis=-1)


def _edge_mlp(g1, g2, tyf, et, we, b1, wdist, ew2, eb2, xwt, xbp):
    return pl.pallas_call(
        _edge_body,
        grid=(g1.shape[0] // BE,),
        in_specs=[
            pl.BlockSpec((BE, TW), lambda i: (i, 0)),
            pl.BlockSpec((BE, TW), lambda i: (i, 0)),
            pl.BlockSpec((BE, 16), lambda i: (i, 0)),
            _full2((2, 128)), _full2((128, HD)),
            _full2((1, HD)), _full2((1, HD)),
            _full2((HD, HD)), _full2((1, HD)),
            _full2((1, HD)), _full2((1, 128)),
        ],
        out_specs=pl.BlockSpec((NCH, BE, 128), lambda i: (0, i, 0)),
        out_shape=jax.ShapeDtypeStruct((NCH, g1.shape[0], 128), f32),
    )(g1, g2, tyf, et, we, b1, wdist, ew2, eb2, xwt, xbp)


def _sc_scatter_body(nsb, m5_h, dst_h, z128_h, hm_h, acc,
                     idx0, idx1, mb0, mb1,
                     lsem0, lsem1, ssem0, ssem1):
    spt = nsb * SB
    c = lax.axis_index("c")
    s = lax.axis_index("s")
    rbase = s * NROWS_T

    slots = ((idx0, mb0, lsem0, ssem0), (idx1, mb1, lsem1, ssem1))

    def fire_load(cid, j, b):
        ib, mb, ls, _ = slots[b]
        pltpu.async_copy(dst_h.at[s * nsb + j], ib, ls)
        pltpu.async_copy(m5_h.at[cid, pl.ds(s * spt + j * SB, SB)], mb, ls)

    def wait_load(b):
        ib, mb, ls, _ = slots[b]
        pltpu.make_async_copy(dst_h.at[0], ib, ls).wait()
        pltpu.make_async_copy(m5_h.at[0, pl.ds(0, SB)], mb, ls).wait()

    def fire_sadd(b):
        ib, mb, _, ss = slots[b]
        pltpu.async_copy(mb, acc.at[ib], ss, add=True)

    def wait_sadd(b):
        ib, mb, _, ss = slots[b]
        pltpu.make_async_copy(mb, acc.at[ib], ss).wait()

    def run_chunk(cid):
        pltpu.sync_copy(z128_h.at[pl.ds(rbase, NROWS_T)],
                        acc.at[pl.ds(rbase, NROWS_T)])
        plsc.subcore_barrier()
        fire_load(cid, 0, 0)

        def body(jj, carry):
            for b in range(2):
                j = jj * 2 + b
                wait_load(b)
                fire_sadd(b)
                if b == 0:
                    @pl.when(jj > 0)
                    def _():
                        wait_sadd(1)
                else:
                    wait_sadd(0)

                @pl.when(j + 1 < nsb)
                def _():
                    fire_load(cid, j + 1, 1 - b)
            return carry

        lax.fori_loop(0, nsb // 2, body, 0)
        wait_sadd(1)
        plsc.subcore_barrier()
        pltpu.sync_copy(acc.at[pl.ds(rbase, NROWS_T)],
                        hm_h.at[cid, pl.ds(rbase, NROWS_T)])
        plsc.subcore_barrier()

    for kk in range(2):
        run_chunk(c * 2 + kk)

    @pl.when(c == 0)
    def _():
        run_chunk(4)


@functools.lru_cache(maxsize=None)
def _build_sc_scatter(ne):
    nsb = (ne // 16) // SB
    return pl.kernel(
        functools.partial(_sc_scatter_body, nsb),
        out_type=jax.ShapeDtypeStruct((NCH, NP, 128), f32),
        mesh=plsc.VectorSubcoreMesh(core_axis_name="c", subcore_axis_name="s"),
        scratch_types=[
            pltpu.VMEM_SHARED((NP, 128), f32),
            pltpu.VMEM((SB,), jnp.int32),
            pltpu.VMEM((SB,), jnp.int32),
            pltpu.VMEM((SB, 128), f32),
            pltpu.VMEM((SB, 128), f32),
            pltpu.SemaphoreType.DMA,
            pltpu.SemaphoreType.DMA,
            pltpu.SemaphoreType.DMA,
            pltpu.SemaphoreType.DMA,
        ],
    )


def _sc_scatter(m5, dst2, z128):
    return _build_sc_scatter(dst2.shape[0] * SB)(m5, dst2, z128)


def _nodeE_body(h, hma, hmb, x, w1h, w1m, hb1, w2, hb2, hout, xout):
    hm5 = hma[...] + hmb[...]
    hm_a = jnp.concatenate([hm5[0], hm5[1], hm5[2], hm5[3]], axis=-1)
    hu = jnp.maximum(jnp.dot(h[...], w1h[...], preferred_element_type=f32)
                     + jnp.dot(hm_a, w1m[...], preferred_element_type=f32)
                     + hb1[...], 0.0)
    hu = jnp.dot(hu, w2[...], preferred_element_type=f32) + hb2[...]
    hout[...] = h[...] + hu
    xout[...] = x[...] + hm5[4][:, :16] * (1.0 / AVG_DEG)


def _node_update(h, hma, hmb, x, w1h, w1m, hb1, w2, hb2):
    return pl.pallas_call(
        _nodeE_body,
        grid=(NP // BN,),
        in_specs=[
            pl.BlockSpec((BN, HD), lambda i: (i, 0)),
            pl.BlockSpec((NCH, BN, 128), lambda i: (0, i, 0)),
            pl.BlockSpec((NCH, BN, 128), lambda i: (0, i, 0)),
            pl.BlockSpec((BN, 16), lambda i: (i, 0)),
            _full2((HD, HD)), _full2((HD, HD)), _full2((1, HD)),
            _full2((HD, HD)), _full2((1, HD)),
        ],
        out_specs=[
            pl.BlockSpec((BN, HD), lambda i: (i, 0)),
            pl.BlockSpec((BN, 16), lambda i: (i, 0)),
        ],
        out_shape=[
            jax.ShapeDtypeStruct((NP, HD), f32),
            jax.ShapeDtypeStruct((NP, 16), f32),
        ],
    )(h, hma, hmb, x, w1h, w1m, hb1, w2, hb2)


def _head_body(h, x, hn, x0, mh, mx, w, bb, eh, ex):
    nh = jnp.dot(h[...], w[...], preferred_element_type=f32) + bb[...]
    eh[...] = (nh - hn[...]) * mh[...]
    ex[...] = (x[...] - x0[...]) * mx[...]


def _head(h, x, hn, x0, mh, mx, w, bb):
    return pl.pallas_call(
        _head_body,
        grid=(NP // BN,),
        in_specs=[
            pl.BlockSpec((BN, HD), lambda i: (i, 0)),
            pl.BlockSpec((BN, 16), lambda i: (i, 0)),
            pl.BlockSpec((BN, INF), lambda i: (i, 0)),
            pl.BlockSpec((BN, 16), lambda i: (i, 0)),
            pl.BlockSpec((BN, INF), lambda i: (i, 0)),
            pl.BlockSpec((BN, 16), lambda i: (i, 0)),
            _full2((HD, INF)), _full2((1, INF)),
        ],
        out_specs=[
            pl.BlockSpec((BN, INF), lambda i: (i, 0)),
            pl.BlockSpec((BN, 16), lambda i: (i, 0)),
        ],
        out_shape=[
            jax.ShapeDtypeStruct((NP, INF), f32),
            jax.ShapeDtypeStruct((NP, 16), f32),
        ],
    )(h, x, hn, x0, mh, mx, w, bb)


def kernel(H_noisy, X_noisy, cond_embedding, edges, edge_types,
           generate_mask, batch_ids, beta, params):
    p = params
    padN = NP - NN
    hn = jnp.pad(H_noisy, ((0, padN), (0, 0)))
    cond = jnp.pad(cond_embedding, ((0, padN), (0, 0)))
    x0 = jnp.pad(X_noisy, ((0, padN), (0, 13)))
    betab = jnp.broadcast_to(jnp.pad(beta, (0, padN))[:, None], (NP, HD // 2))
    maskf = jnp.pad(generate_mask, (0, padN)).astype(f32)
    mh = jnp.broadcast_to(maskf[:, None], (NP, INF))
    mx = jnp.broadcast_to(maskf[:, None], (NP, 16))

    dstp = jnp.pad(edges[1], (0, EP - EDGES), constant_values=NN)
    srcp = jnp.pad(edges[0], (0, EP - EDGES), constant_values=NN)
    EPH = EP // 2
    dstgA = dstp[:EPH].reshape(EPH // GB, GB)
    dstgB = dstp[EPH:].reshape(EPH // GB, GB)
    srcgA = srcp[:EPH].reshape(EPH // GB, GB)
    srcgB = srcp[EPH:].reshape(EPH // GB, GB)
    dstsA = dstp[:EPH].reshape(EPH // SB, SB)
    dstsB = dstp[EPH:].reshape(EPH // SB, SB)
    tyf = jnp.broadcast_to(
        jnp.pad(edge_types, (0, EP - EDGES)).astype(f32)[:, None], (EP, 16))
    tyfA = tyf[:EP // 2]
    tyfB = tyf[EP // 2:]
    z128 = jnp.zeros((NP, 128), f32)

    w1 = p['inp_w1']
    h = _input_mlp(hn, cond, betab,
                   w1[0:INF], w1[INF:INF + HD],
                   w1[INF + HD:INF + HD + HD // 2], w1[INF + HD + HD // 2:],
                   p['inp_b1'].reshape(1, HD),
                   p['inp_w2'], p['inp_b2'].reshape(1, HD),
                   p['inp_w3'], p['inp_b3'].reshape(1, HD))
    x = x0

    for l in range(NLAYERS):
        ew1 = p[f'l{l}_ew1']
        wd = ew1[0:HD]
        ws = ew1[HD:2 * HD]
        wdist = ew1[2 * HD:2 * HD + 1].reshape(1, HD)
        we = ew1[2 * HD + 1:]
        xbp = jnp.pad(p[f'l{l}_xb'].reshape(1, 1), ((0, 0), (0, 127)))

        ad, as_ = _node_pre(h, x, wd, ws)
        ew_args = (p['edge_table'], we,
                   p[f'l{l}_eb1'].reshape(1, HD), wdist,
                   p[f'l{l}_ew2'], p[f'l{l}_eb2'].reshape(1, HD),
                   p[f'l{l}_xw'].reshape(1, HD), xbp)
        g1a, g2a = _sc_gather(ad, as_, dstgA, srcgA)
        g1b, g2b = _sc_gather(ad, as_, dstgB, srcgB)
        m5a = _edge_mlp(g1a, g2a, tyfA, *ew_args)
        m5b = _edge_mlp(g1b, g2b, tyfB, *ew_args)
        hma = _sc_scatter(m5a, dstsA, z128)
        hmb = _sc_scatter(m5b, dstsB, z128)
        hw1 = p[f'l{l}_hw1']
        h, x = _node_update(h, hma, hmb, x,
                            hw1[0:HD], hw1[HD:],
                            p[f'l{l}_hb1'].reshape(1, HD),
                            p[f'l{l}_hw2'], p[f'l{l}_hb2'].reshape(1, HD))

    eh, ex = _head(h, x, hn, x0, mh, mx,
                   p['h2i_w'], p['h2i_b'].reshape(1, INF))
    return eh[:NN], ex[:NN, :3]


# speedup vs baseline: 2.5424x; 1.1216x over previous
import functools

import numpy as np
import jax
import jax.numpy as jnp
from jax import lax
from jax.experimental import pallas as pl
from jax.experimental.pallas import tpu as pltpu
from jax.experimental.pallas import tpu_sc as plsc

NN = 10000
NP = 10240
EDGES = 160000
EP = 163840
HD = 512
INF = 256
NLAYERS = 3
AVG_DEG = 16.0

BN = 256
BE = 512
NW = 32
GB = 32
NGB = (EP // NW) // GB
SB = 128
SPT = EP // 16
NSB = SPT // SB
NROWS_T = NP // 16

TW = 640
NCH = 5

f32 = jnp.float32


def _inp_body(hn, cond, betab, w1h, w1c, w1ts, w1tc, b1, w2, b2, w3, b3, out):
    half = HD // 2
    k = lax.broadcasted_iota(jnp.int32, (1, half), 1).astype(f32)
    freqs = jnp.exp(-np.log(10000.0) * k / (half - 1))
    args = betab[...] * freqs
    sn = jnp.sin(args)
    cs = jnp.cos(args)
    acc = (jnp.dot(hn[...], w1h[...], preferred_element_type=f32)
           + jnp.dot(cond[...], w1c[...], preferred_element_type=f32)
           + jnp.dot(sn, w1ts[...], preferred_element_type=f32)
           + jnp.dot(cs, w1tc[...], preferred_element_type=f32)
           + b1[...])
    h = jnp.maximum(acc, 0.0)
    h = jnp.maximum(jnp.dot(h, w2[...], preferred_element_type=f32) + b2[...], 0.0)
    out[...] = jnp.dot(h, w3[...], preferred_element_type=f32) + b3[...]


def _full2(shape):
    return pl.BlockSpec(shape, lambda i: (0, 0))


def _input_mlp(hn, cond, betab, w1h, w1c, w1ts, w1tc, b1, w2, b2, w3, b3):
    return pl.pallas_call(
        _inp_body,
        grid=(NP // BN,),
        in_specs=[
            pl.BlockSpec((BN, INF), lambda i: (i, 0)),
            pl.BlockSpec((BN, HD), lambda i: (i, 0)),
            pl.BlockSpec((BN, HD // 2), lambda i: (i, 0)),
            _full2((INF, HD)), _full2((HD, HD)),
            _full2((HD // 2, HD)), _full2((HD // 2, HD)), _full2((1, HD)),
            _full2((HD, HD)), _full2((1, HD)),
            _full2((HD, HD)), _full2((1, HD)),
        ],
        out_specs=pl.BlockSpec((BN, HD), lambda i: (i, 0)),
        out_shape=jax.ShapeDtypeStruct((NP, HD), f32),
    )(hn, cond, betab, w1h, w1c, w1ts, w1tc, b1, w2, b2, w3, b3)


def _nodeA_body(h, x, wd, ws, ad, as_):
    z = jnp.zeros((BN, TW - HD - 16), f32)
    ad[...] = jnp.concatenate(
        [jnp.dot(h[...], wd[...], preferred_element_type=f32), x[...], z], axis=-1)
    as_[...] = jnp.concatenate(
        [jnp.dot(h[...], ws[...], preferred_element_type=f32), x[...], z], axis=-1)


def _node_pre(h, x, wd, ws):
    return pl.pallas_call(
        _nodeA_body,
        grid=(NP // BN,),
        in_specs=[
            pl.BlockSpec((BN, HD), lambda i: (i, 0)),
            pl.BlockSpec((BN, 16), lambda i: (i, 0)),
            _full2((HD, HD)), _full2((HD, HD)),
        ],
        out_specs=[
            pl.BlockSpec((BN, TW), lambda i: (i, 0)),
            pl.BlockSpec((BN, TW), lambda i: (i, 0)),
        ],
        out_shape=[
            jax.ShapeDtypeStruct((NP, TW), f32),
            jax.ShapeDtypeStruct((NP, TW), f32),
        ],
    )(h, x, wd, ws)


def _sc_gather_body(ngb, ad_h, as_h, dst_h, src_h, g1_h,
                    idxd, idxs, bufd0, bufs0, bufd1, bufs1,
                    gsem0, gsem1, wsem0, wsem1):
    wid = lax.axis_index("s") * 2 + lax.axis_index("c")
    rbase = wid * ngb
    pltpu.sync_copy(dst_h.at[pl.ds(rbase, ngb)], idxd)
    pltpu.sync_copy(src_h.at[pl.ds(rbase, ngb)], idxs)

    slots = ((bufd0, bufs0, gsem0, wsem0), (bufd1, bufs1, gsem1, wsem1))

    def fire_gather(j, b):
        bd, bs, gs, _ = slots[b]
        pltpu.async_copy(ad_h.at[idxd.at[j]], bd, gs)
        pltpu.async_copy(as_h.at[idxs.at[j]], bs, gs)

    def wait_gather(b):
        bd, bs, gs, _ = slots[b]
        pltpu.make_async_copy(ad_h.at[idxd.at[0]], bd, gs).wait()
        pltpu.make_async_copy(as_h.at[idxs.at[0]], bs, gs).wait()

    def tec_add(b):
        bd, bs, _, _ = slots[b]

        def row(i, carry):
            for k in range(HD // 16):
                sl = pl.ds(k * 16, 16)
                bd[i, sl] = bd[i, sl] + bs[i, sl]
            bd[i, pl.ds(HD + 16, 16)] = bs[i, pl.ds(HD, 16)]
            return carry

        lax.fori_loop(0, GB, row, 0)

    def fire_wb(j, b):
        bd, bs, _, ws = slots[b]
        base = (rbase + j) * GB
        pltpu.async_copy(bd, g1_h.at[pl.ds(base, GB)], ws)

    def wait_wb(b):
        bd, bs, _, ws = slots[b]
        pltpu.make_async_copy(bd, g1_h.at[pl.ds(0, GB)], ws).wait()

    fire_gather(0, 0)

    def body(jj, carry):
        for b in range(2):
            j = jj * 2 + b
            wait_gather(b)
            tec_add(b)
            fire_wb(j, b)
            if b == 0:
                @pl.when(jj > 0)
                def _():
                    wait_wb(1)
            else:
                wait_wb(0)

            @pl.when(j + 1 < ngb)
            def _():
                fire_gather(j + 1, 1 - b)
        return carry

    lax.fori_loop(0, ngb // 2, body, 0)
    wait_wb(1)


@functools.lru_cache(maxsize=None)
def _build_sc_gather(ne):
    ngb = (ne // NW) // GB
    return pl.kernel(
        functools.partial(_sc_gather_body, ngb),
        out_type=jax.ShapeDtypeStruct((ne, TW), f32),
        mesh=plsc.VectorSubcoreMesh(core_axis_name="c", subcore_axis_name="s"),
        scratch_types=[
            pltpu.VMEM((ngb, GB), jnp.int32),
            pltpu.VMEM((ngb, GB), jnp.int32),
            pltpu.VMEM((GB, TW), f32),
            pltpu.VMEM((GB, TW), f32),
            pltpu.VMEM((GB, TW), f32),
            pltpu.VMEM((GB, TW), f32),
            pltpu.SemaphoreType.DMA,
            pltpu.SemaphoreType.DMA,
            pltpu.SemaphoreType.DMA,
            pltpu.SemaphoreType.DMA,
        ],
    )


def _sc_gather(ad, as_, dst2, src2):
    return _build_sc_gather(dst2.shape[0] * GB)(ad, as_, dst2, src2)


def _edge_body(g1, tyf, et, we, b1, wdist, ew2, eb2, xwt, xbp, mout):
    tvec = jnp.dot(et[...], we[...], preferred_element_type=f32)
    tcol = tyf[...][:, 0:1]
    ga = g1[...]
    rel = ga[:, HD:HD + 16] - ga[:, HD + 16:HD + 32]
    d2 = jnp.sum(rel * rel, axis=1, keepdims=True)
    sel = tvec[0:1, :] + tcol * (tvec[1:2, :] - tvec[0:1, :])
    d2b = d2.astype(jnp.bfloat16).astype(f32)
    wdb = wdist[...].astype(jnp.bfloat16).astype(f32)
    pre = ga[:, :HD] + d2b * wdb + sel + b1[...]
    a1 = jnp.maximum(pre, 0.0)
    m = jnp.maximum(jnp.dot(a1, ew2[...], preferred_element_type=f32) + eb2[...], 0.0)
    mb = m.astype(jnp.bfloat16).astype(f32)
    xwb = xwt[...].astype(jnp.bfloat16).astype(f32)
    coef = jnp.sum(mb * xwb, axis=1, keepdims=True) + xbp[0, 0]
    relc = rel * coef
    for c in range(4):
        mout[c, :, :] = m[:, c * 128:(c + 1) * 128]
    mout[4, :, :] = jnp.concatenate([relc, jnp.zeros((BE, 112), f32)], axis=-1)


def _edge_mlp(g1, tyf, et, we, b1, wdist, ew2, eb2, xwt, xbp):
    return pl.pallas_call(
        _edge_body,
        grid=(g1.shape[0] // BE,),
        in_specs=[
            pl.BlockSpec((BE, TW), lambda i: (i, 0)),
            pl.BlockSpec((BE, 16), lambda i: (i, 0)),
            _full2((2, 128)), _full2((128, HD)),
            _full2((1, HD)), _full2((1, HD)),
            _full2((HD, HD)), _full2((1, HD)),
            _full2((1, HD)), _full2((1, 128)),
        ],
        out_specs=pl.BlockSpec((NCH, BE, 128), lambda i: (0, i, 0)),
        out_shape=jax.ShapeDtypeStruct((NCH, g1.shape[0], 128), f32),
    )(g1, tyf, et, we, b1, wdist, ew2, eb2, xwt, xbp)


def _sc_scatter_body(nsb, m5_h, dst_h, z128_h, hm_h, acc,
                     idx0, idx1, mb0, mb1,
                     lsem0, lsem1, ssem0, ssem1):
    spt = nsb * SB
    c = lax.axis_index("c")
    s = lax.axis_index("s")
    rbase = s * NROWS_T

    slots = ((idx0, mb0, lsem0, ssem0), (idx1, mb1, lsem1, ssem1))

    def fire_load(cid, j, b):
        ib, mb, ls, _ = slots[b]
        pltpu.async_copy(dst_h.at[s * nsb + j], ib, ls)
        pltpu.async_copy(m5_h.at[cid, pl.ds(s * spt + j * SB, SB)], mb, ls)

    def wait_load(b):
        ib, mb, ls, _ = slots[b]
        pltpu.make_async_copy(dst_h.at[0], ib, ls).wait()
        pltpu.make_async_copy(m5_h.at[0, pl.ds(0, SB)], mb, ls).wait()

    def fire_sadd(b):
        ib, mb, _, ss = slots[b]
        pltpu.async_copy(mb, acc.at[ib], ss, add=True)

    def wait_sadd(b):
        ib, mb, _, ss = slots[b]
        pltpu.make_async_copy(mb, acc.at[ib], ss).wait()

    def run_chunk(cid):
        pltpu.sync_copy(z128_h.at[pl.ds(rbase, NROWS_T)],
                        acc.at[pl.ds(rbase, NROWS_T)])
        plsc.subcore_barrier()
        fire_load(cid, 0, 0)

        def body(jj, carry):
            for b in range(2):
                j = jj * 2 + b
                wait_load(b)
                fire_sadd(b)
                if b == 0:
                    @pl.when(jj > 0)
                    def _():
                        wait_sadd(1)
                else:
                    wait_sadd(0)

                @pl.when(j + 1 < nsb)
                def _():
                    fire_load(cid, j + 1, 1 - b)
            return carry

        lax.fori_loop(0, nsb // 2, body, 0)
        wait_sadd(1)
        plsc.subcore_barrier()
        pltpu.sync_copy(acc.at[pl.ds(rbase, NROWS_T)],
                        hm_h.at[cid, pl.ds(rbase, NROWS_T)])
        plsc.subcore_barrier()

    for kk in range(2):
        run_chunk(c * 2 + kk)

    @pl.when(c == 0)
    def _():
        run_chunk(4)


@functools.lru_cache(maxsize=None)
def _build_sc_scatter(ne):
    nsb = (ne // 16) // SB
    return pl.kernel(
        functools.partial(_sc_scatter_body, nsb),
        out_type=jax.ShapeDtypeStruct((NCH, NP, 128), f32),
        mesh=plsc.VectorSubcoreMesh(core_axis_name="c", subcore_axis_name="s"),
        scratch_types=[
            pltpu.VMEM_SHARED((NP, 128), f32),
            pltpu.VMEM((SB,), jnp.int32),
            pltpu.VMEM((SB,), jnp.int32),
            pltpu.VMEM((SB, 128), f32),
            pltpu.VMEM((SB, 128), f32),
            pltpu.SemaphoreType.DMA,
            pltpu.SemaphoreType.DMA,
            pltpu.SemaphoreType.DMA,
            pltpu.SemaphoreType.DMA,
        ],
    )


def _sc_scatter(m5, dst2, z128):
    return _build_sc_scatter(dst2.shape[0] * SB)(m5, dst2, z128)


def _nodeE_body(h, hma, hmb, x, w1h, w1m, hb1, w2, hb2, hout, xout):
    hm5 = hma[...] + hmb[...]
    hm_a = jnp.concatenate([hm5[0], hm5[1], hm5[2], hm5[3]], axis=-1)
    hu = jnp.maximum(jnp.dot(h[...], w1h[...], preferred_element_type=f32)
                     + jnp.dot(hm_a, w1m[...], preferred_element_type=f32)
                     + hb1[...], 0.0)
    hu = jnp.dot(hu, w2[...], preferred_element_type=f32) + hb2[...]
    hout[...] = h[...] + hu
    xout[...] = x[...] + hm5[4][:, :16] * (1.0 / AVG_DEG)


def _node_update(h, hma, hmb, x, w1h, w1m, hb1, w2, hb2):
    return pl.pallas_call(
        _nodeE_body,
        grid=(NP // BN,),
        in_specs=[
            pl.BlockSpec((BN, HD), lambda i: (i, 0)),
            pl.BlockSpec((NCH, BN, 128), lambda i: (0, i, 0)),
            pl.BlockSpec((NCH, BN, 128), lambda i: (0, i, 0)),
            pl.BlockSpec((BN, 16), lambda i: (i, 0)),
            _full2((HD, HD)), _full2((HD, HD)), _full2((1, HD)),
            _full2((HD, HD)), _full2((1, HD)),
        ],
        out_specs=[
            pl.BlockSpec((BN, HD), lambda i: (i, 0)),
            pl.BlockSpec((BN, 16), lambda i: (i, 0)),
        ],
        out_shape=[
            jax.ShapeDtypeStruct((NP, HD), f32),
            jax.ShapeDtypeStruct((NP, 16), f32),
        ],
    )(h, hma, hmb, x, w1h, w1m, hb1, w2, hb2)


def _head_body(h, x, hn, x0, mh, mx, w, bb, eh, ex):
    nh = jnp.dot(h[...], w[...], preferred_element_type=f32) + bb[...]
    eh[...] = (nh - hn[...]) * mh[...]
    ex[...] = (x[...] - x0[...]) * mx[...]


def _head(h, x, hn, x0, mh, mx, w, bb):
    return pl.pallas_call(
        _head_body,
        grid=(NP // BN,),
        in_specs=[
            pl.BlockSpec((BN, HD), lambda i: (i, 0)),
            pl.BlockSpec((BN, 16), lambda i: (i, 0)),
            pl.BlockSpec((BN, INF), lambda i: (i, 0)),
            pl.BlockSpec((BN, 16), lambda i: (i, 0)),
            pl.BlockSpec((BN, INF), lambda i: (i, 0)),
            pl.BlockSpec((BN, 16), lambda i: (i, 0)),
            _full2((HD, INF)), _full2((1, INF)),
        ],
        out_specs=[
            pl.BlockSpec((BN, INF), lambda i: (i, 0)),
            pl.BlockSpec((BN, 16), lambda i: (i, 0)),
        ],
        out_shape=[
            jax.ShapeDtypeStruct((NP, INF), f32),
            jax.ShapeDtypeStruct((NP, 16), f32),
        ],
    )(h, x, hn, x0, mh, mx, w, bb)


def kernel(H_noisy, X_noisy, cond_embedding, edges, edge_types,
           generate_mask, batch_ids, beta, params):
    p = params
    padN = NP - NN
    hn = jnp.pad(H_noisy, ((0, padN), (0, 0)))
    cond = jnp.pad(cond_embedding, ((0, padN), (0, 0)))
    x0 = jnp.pad(X_noisy, ((0, padN), (0, 13)))
    betab = jnp.broadcast_to(jnp.pad(beta, (0, padN))[:, None], (NP, HD // 2))
    maskf = jnp.pad(generate_mask, (0, padN)).astype(f32)
    mh = jnp.broadcast_to(maskf[:, None], (NP, INF))
    mx = jnp.broadcast_to(maskf[:, None], (NP, 16))

    dstp = jnp.pad(edges[1], (0, EP - EDGES), constant_values=NN)
    srcp = jnp.pad(edges[0], (0, EP - EDGES), constant_values=NN)
    EPH = EP // 2
    dstgA = dstp[:EPH].reshape(EPH // GB, GB)
    dstgB = dstp[EPH:].reshape(EPH // GB, GB)
    srcgA = srcp[:EPH].reshape(EPH // GB, GB)
    srcgB = srcp[EPH:].reshape(EPH // GB, GB)
    dstsA = dstp[:EPH].reshape(EPH // SB, SB)
    dstsB = dstp[EPH:].reshape(EPH // SB, SB)
    tyf = jnp.broadcast_to(
        jnp.pad(edge_types, (0, EP - EDGES)).astype(f32)[:, None], (EP, 16))
    tyfA = tyf[:EP // 2]
    tyfB = tyf[EP // 2:]
    z128 = jnp.zeros((NP, 128), f32)

    w1 = p['inp_w1']
    h = _input_mlp(hn, cond, betab,
                   w1[0:INF], w1[INF:INF + HD],
                   w1[INF + HD:INF + HD + HD // 2], w1[INF + HD + HD // 2:],
                   p['inp_b1'].reshape(1, HD),
                   p['inp_w2'], p['inp_b2'].reshape(1, HD),
                   p['inp_w3'], p['inp_b3'].reshape(1, HD))
    x = x0

    for l in range(NLAYERS):
        ew1 = p[f'l{l}_ew1']
        wd = ew1[0:HD]
        ws = ew1[HD:2 * HD]
        wdist = ew1[2 * HD:2 * HD + 1].reshape(1, HD)
        we = ew1[2 * HD + 1:]
        xbp = jnp.pad(p[f'l{l}_xb'].reshape(1, 1), ((0, 0), (0, 127)))

        ad, as_ = _node_pre(h, x, wd, ws)
        ew_args = (p['edge_table'], we,
                   p[f'l{l}_eb1'].reshape(1, HD), wdist,
                   p[f'l{l}_ew2'], p[f'l{l}_eb2'].reshape(1, HD),
                   p[f'l{l}_xw'].reshape(1, HD), xbp)
        g1a = _sc_gather(ad, as_, dstgA, srcgA)
        g1b = _sc_gather(ad, as_, dstgB, srcgB)
        m5a = _edge_mlp(g1a, tyfA, *ew_args)
        m5b = _edge_mlp(g1b, tyfB, *ew_args)
        hma = _sc_scatter(m5a, dstsA, z128)
        hmb = _sc_scatter(m5b, dstsB, z128)
        hw1 = p[f'l{l}_hw1']
        h, x = _node_update(h, hma, hmb, x,
                            hw1[0:HD], hw1[HD:],
                            p[f'l{l}_hb1'].reshape(1, HD),
                            p[f'l{l}_hw2'], p[f'l{l}_hb2'].reshape(1, HD))

    eh, ex = _head(h, x, hn, x0, mh, mx,
                   p['h2i_w'], p['h2i_b'].reshape(1, INF))
    return eh[:NN], ex[:NN, :3]


# speedup vs baseline: 2.5550x; 1.0049x over previous
import functools

import numpy as np
import jax
import jax.numpy as jnp
from jax import lax
from jax.experimental import pallas as pl
from jax.experimental.pallas import tpu as pltpu
from jax.experimental.pallas import tpu_sc as plsc

NN = 10000
NP = 10240
EDGES = 160000
EP = 163840
HD = 512
INF = 256
NLAYERS = 3
AVG_DEG = 16.0

BN = 256
BE = 512
NW = 32
GB = 40
NGB = (EP // NW) // GB
SB = 128
SPT = EP // 16
NSB = SPT // SB
NROWS_T = NP // 16

TW = 640
NCH = 5

f32 = jnp.float32


def _inp_body(hn, cond, betab, w1h, w1c, w1ts, w1tc, b1, w2, b2, w3, b3, out):
    half = HD // 2
    k = lax.broadcasted_iota(jnp.int32, (1, half), 1).astype(f32)
    freqs = jnp.exp(-np.log(10000.0) * k / (half - 1))
    args = betab[...] * freqs
    sn = jnp.sin(args)
    cs = jnp.cos(args)
    acc = (jnp.dot(hn[...], w1h[...], preferred_element_type=f32)
           + jnp.dot(cond[...], w1c[...], preferred_element_type=f32)
           + jnp.dot(sn, w1ts[...], preferred_element_type=f32)
           + jnp.dot(cs, w1tc[...], preferred_element_type=f32)
           + b1[...])
    h = jnp.maximum(acc, 0.0)
    h = jnp.maximum(jnp.dot(h, w2[...], preferred_element_type=f32) + b2[...], 0.0)
    out[...] = jnp.dot(h, w3[...], preferred_element_type=f32) + b3[...]


def _full2(shape):
    return pl.BlockSpec(shape, lambda i: (0, 0))


def _input_mlp(hn, cond, betab, w1h, w1c, w1ts, w1tc, b1, w2, b2, w3, b3):
    return pl.pallas_call(
        _inp_body,
        grid=(NP // BN,),
        in_specs=[
            pl.BlockSpec((BN, INF), lambda i: (i, 0)),
            pl.BlockSpec((BN, HD), lambda i: (i, 0)),
            pl.BlockSpec((BN, HD // 2), lambda i: (i, 0)),
            _full2((INF, HD)), _full2((HD, HD)),
            _full2((HD // 2, HD)), _full2((HD // 2, HD)), _full2((1, HD)),
            _full2((HD, HD)), _full2((1, HD)),
            _full2((HD, HD)), _full2((1, HD)),
        ],
        out_specs=pl.BlockSpec((BN, HD), lambda i: (i, 0)),
        out_shape=jax.ShapeDtypeStruct((NP, HD), f32),
    )(hn, cond, betab, w1h, w1c, w1ts, w1tc, b1, w2, b2, w3, b3)


def _nodeA_body(h, x, wd, ws, ad, as_):
    z = jnp.zeros((BN, TW - HD - 16), f32)
    ad[...] = jnp.concatenate(
        [jnp.dot(h[...], wd[...], preferred_element_type=f32), x[...], z], axis=-1)
    as_[...] = jnp.concatenate(
        [jnp.dot(h[...], ws[...], preferred_element_type=f32), x[...], z], axis=-1)


def _node_pre(h, x, wd, ws):
    return pl.pallas_call(
        _nodeA_body,
        grid=(NP // BN,),
        in_specs=[
            pl.BlockSpec((BN, HD), lambda i: (i, 0)),
            pl.BlockSpec((BN, 16), lambda i: (i, 0)),
            _full2((HD, HD)), _full2((HD, HD)),
        ],
        out_specs=[
            pl.BlockSpec((BN, TW), lambda i: (i, 0)),
            pl.BlockSpec((BN, TW), lambda i: (i, 0)),
        ],
        out_shape=[
            jax.ShapeDtypeStruct((NP, TW), f32),
            jax.ShapeDtypeStruct((NP, TW), f32),
        ],
    )(h, x, wd, ws)


def _sc_gather_body(ngb, ad_h, as_h, dst_h, src_h, g1_h,
                    idxd, idxs, bufd0, bufs0, bufd1, bufs1,
                    gsem0, gsem1, wsem0, wsem1):
    wid = lax.axis_index("s") * 2 + lax.axis_index("c")
    rbase = wid * ngb
    pltpu.sync_copy(dst_h.at[pl.ds(rbase, ngb)], idxd)
    pltpu.sync_copy(src_h.at[pl.ds(rbase, ngb)], idxs)

    slots = ((bufd0, bufs0, gsem0, wsem0), (bufd1, bufs1, gsem1, wsem1))

    def fire_gather(j, b):
        bd, bs, gs, _ = slots[b]
        pltpu.async_copy(ad_h.at[idxd.at[j]], bd, gs)
        pltpu.async_copy(as_h.at[idxs.at[j]], bs, gs)

    def wait_gather(b):
        bd, bs, gs, _ = slots[b]
        pltpu.make_async_copy(ad_h.at[idxd.at[0]], bd, gs).wait()
        pltpu.make_async_copy(as_h.at[idxs.at[0]], bs, gs).wait()

    def tec_add(b):
        bd, bs, _, _ = slots[b]

        def row(i, carry):
            for k in range(HD // 16):
                sl = pl.ds(k * 16, 16)
                bd[i, sl] = bd[i, sl] + bs[i, sl]
            bd[i, pl.ds(HD + 16, 16)] = bs[i, pl.ds(HD, 16)]
            return carry

        lax.fori_loop(0, GB, row, 0)

    def fire_wb(j, b):
        bd, bs, _, ws = slots[b]
        base = (rbase + j) * GB
        pltpu.async_copy(bd, g1_h.at[pl.ds(base, GB)], ws)

    def wait_wb(b):
        bd, bs, _, ws = slots[b]
        pltpu.make_async_copy(bd, g1_h.at[pl.ds(0, GB)], ws).wait()

    fire_gather(0, 0)

    def body(jj, carry):
        for b in range(2):
            j = jj * 2 + b
            wait_gather(b)
            tec_add(b)
            fire_wb(j, b)
            if b == 0:
                @pl.when(jj > 0)
                def _():
                    wait_wb(1)
            else:
                wait_wb(0)

            @pl.when(j + 1 < ngb)
            def _():
                fire_gather(j + 1, 1 - b)
        return carry

    lax.fori_loop(0, ngb // 2, body, 0)
    wait_wb(1)


@functools.lru_cache(maxsize=None)
def _build_sc_gather(ne):
    ngb = (ne // NW) // GB
    return pl.kernel(
        functools.partial(_sc_gather_body, ngb),
        out_type=jax.ShapeDtypeStruct((ne, TW), f32),
        mesh=plsc.VectorSubcoreMesh(core_axis_name="c", subcore_axis_name="s"),
        scratch_types=[
            pltpu.VMEM((ngb, GB), jnp.int32),
            pltpu.VMEM((ngb, GB), jnp.int32),
            pltpu.VMEM((GB, TW), f32),
            pltpu.VMEM((GB, TW), f32),
            pltpu.VMEM((GB, TW), f32),
            pltpu.VMEM((GB, TW), f32),
            pltpu.SemaphoreType.DMA,
            pltpu.SemaphoreType.DMA,
            pltpu.SemaphoreType.DMA,
            pltpu.SemaphoreType.DMA,
        ],
    )


def _sc_gather(ad, as_, dst2, src2):
    return _build_sc_gather(dst2.shape[0] * GB)(ad, as_, dst2, src2)


def _edge_body(g1, tyf, et, we, b1, wdist, ew2, eb2, xwt, xbp, mout):
    tvec = jnp.dot(et[...], we[...], preferred_element_type=f32)
    tcol = tyf[...][:, 0:1]
    ga = g1[...]
    rel = ga[:, HD:HD + 16] - ga[:, HD + 16:HD + 32]
    d2 = jnp.sum(rel * rel, axis=1, keepdims=True)
    sel = tvec[0:1, :] + tcol * (tvec[1:2, :] - tvec[0:1, :])
    d2b = d2.astype(jnp.bfloat16).astype(f32)
    wdb = wdist[...].astype(jnp.bfloat16).astype(f32)
    pre = ga[:, :HD] + d2b * wdb + sel + b1[...]
    a1 = jnp.maximum(pre, 0.0)
    m = jnp.maximum(jnp.dot(a1, ew2[...], preferred_element_type=f32) + eb2[...], 0.0)
    mb = m.astype(jnp.bfloat16).astype(f32)
    xwb = xwt[...].astype(jnp.bfloat16).astype(f32)
    coef = jnp.sum(mb * xwb, axis=1, keepdims=True) + xbp[0, 0]
    relc = rel * coef
    for c in range(4):
        mout[c, :, :] = m[:, c * 128:(c + 1) * 128]
    mout[4, :, :] = jnp.concatenate([relc, jnp.zeros((BE, 112), f32)], axis=-1)


def _edge_mlp(g1, tyf, et, we, b1, wdist, ew2, eb2, xwt, xbp):
    return pl.pallas_call(
        _edge_body,
        grid=(g1.shape[0] // BE,),
        in_specs=[
            pl.BlockSpec((BE, TW), lambda i: (i, 0)),
            pl.BlockSpec((BE, 16), lambda i: (i, 0)),
            _full2((2, 128)), _full2((128, HD)),
            _full2((1, HD)), _full2((1, HD)),
            _full2((HD, HD)), _full2((1, HD)),
            _full2((1, HD)), _full2((1, 128)),
        ],
        out_specs=pl.BlockSpec((NCH, BE, 128), lambda i: (0, i, 0)),
        out_shape=jax.ShapeDtypeStruct((NCH, g1.shape[0], 128), f32),
    )(g1, tyf, et, we, b1, wdist, ew2, eb2, xwt, xbp)


def _sc_scatter_body(nsb, m5_h, dst_h, z128_h, hm_h, acc,
                     idx0, idx1, mb0, mb1,
                     lsem0, lsem1, ssem0, ssem1):
    spt = nsb * SB
    c = lax.axis_index("c")
    s = lax.axis_index("s")
    rbase = s * NROWS_T

    slots = ((idx0, mb0, lsem0, ssem0), (idx1, mb1, lsem1, ssem1))

    def fire_load(cid, j, b):
        ib, mb, ls, _ = slots[b]
        pltpu.async_copy(dst_h.at[s * nsb + j], ib, ls)
        pltpu.async_copy(m5_h.at[cid, pl.ds(s * spt + j * SB, SB)], mb, ls)

    def wait_load(b):
        ib, mb, ls, _ = slots[b]
        pltpu.make_async_copy(dst_h.at[0], ib, ls).wait()
        pltpu.make_async_copy(m5_h.at[0, pl.ds(0, SB)], mb, ls).wait()

    def fire_sadd(b):
        ib, mb, _, ss = slots[b]
        pltpu.async_copy(mb, acc.at[ib], ss, add=True)

    def wait_sadd(b):
        ib, mb, _, ss = slots[b]
        pltpu.make_async_copy(mb, acc.at[ib], ss).wait()

    def run_chunk(cid):
        pltpu.sync_copy(z128_h.at[pl.ds(rbase, NROWS_T)],
                        acc.at[pl.ds(rbase, NROWS_T)])
        plsc.subcore_barrier()
        fire_load(cid, 0, 0)

        def body(jj, carry):
            for b in range(2):
                j = jj * 2 + b
                wait_load(b)
                fire_sadd(b)
                if b == 0:
                    @pl.when(jj > 0)
                    def _():
                        wait_sadd(1)
                else:
                    wait_sadd(0)

                @pl.when(j + 1 < nsb)
                def _():
                    fire_load(cid, j + 1, 1 - b)
            return carry

        lax.fori_loop(0, nsb // 2, body, 0)
        wait_sadd(1)
        plsc.subcore_barrier()
        pltpu.sync_copy(acc.at[pl.ds(rbase, NROWS_T)],
                        hm_h.at[cid, pl.ds(rbase, NROWS_T)])
        plsc.subcore_barrier()

    for kk in range(2):
        run_chunk(c * 2 + kk)

    @pl.when(c == 0)
    def _():
        run_chunk(4)


@functools.lru_cache(maxsize=None)
def _build_sc_scatter(ne):
    nsb = (ne // 16) // SB
    return pl.kernel(
        functools.partial(_sc_scatter_body, nsb),
        out_type=jax.ShapeDtypeStruct((NCH, NP, 128), f32),
        mesh=plsc.VectorSubcoreMesh(core_axis_name="c", subcore_axis_name="s"),
        scratch_types=[
            pltpu.VMEM_SHARED((NP, 128), f32),
            pltpu.VMEM((SB,), jnp.int32),
            pltpu.VMEM((SB,), jnp.int32),
            pltpu.VMEM((SB, 128), f32),
            pltpu.VMEM((SB, 128), f32),
            pltpu.SemaphoreType.DMA,
            pltpu.SemaphoreType.DMA,
            pltpu.SemaphoreType.DMA,
            pltpu.SemaphoreType.DMA,
        ],
    )


def _sc_scatter(m5, dst2, z128):
    return _build_sc_scatter(dst2.shape[0] * SB)(m5, dst2, z128)


def _nodeE_body(h, hma, hmb, x, w1h, w1m, hb1, w2, hb2, hout, xout):
    hm5 = hma[...] + hmb[...]
    hm_a = jnp.concatenate([hm5[0], hm5[1], hm5[2], hm5[3]], axis=-1)
    hu = jnp.maximum(jnp.dot(h[...], w1h[...], preferred_element_type=f32)
                     + jnp.dot(hm_a, w1m[...], preferred_element_type=f32)
                     + hb1[...], 0.0)
    hu = jnp.dot(hu, w2[...], preferred_element_type=f32) + hb2[...]
    hout[...] = h[...] + hu
    xout[...] = x[...] + hm5[4][:, :16] * (1.0 / AVG_DEG)


def _node_update(h, hma, hmb, x, w1h, w1m, hb1, w2, hb2):
    return pl.pallas_call(
        _nodeE_body,
        grid=(NP // BN,),
        in_specs=[
            pl.BlockSpec((BN, HD), lambda i: (i, 0)),
            pl.BlockSpec((NCH, BN, 128), lambda i: (0, i, 0)),
            pl.BlockSpec((NCH, BN, 128), lambda i: (0, i, 0)),
            pl.BlockSpec((BN, 16), lambda i: (i, 0)),
            _full2((HD, HD)), _full2((HD, HD)), _full2((1, HD)),
            _full2((HD, HD)), _full2((1, HD)),
        ],
        out_specs=[
            pl.BlockSpec((BN, HD), lambda i: (i, 0)),
            pl.BlockSpec((BN, 16), lambda i: (i, 0)),
        ],
        out_shape=[
            jax.ShapeDtypeStruct((NP, HD), f32),
            jax.ShapeDtypeStruct((NP, 16), f32),
        ],
    )(h, hma, hmb, x, w1h, w1m, hb1, w2, hb2)


def _head_body(h, x, hn, x0, mh, mx, w, bb, eh, ex):
    nh = jnp.dot(h[...], w[...], preferred_element_type=f32) + bb[...]
    eh[...] = (nh - hn[...]) * mh[...]
    ex[...] = (x[...] - x0[...]) * mx[...]


def _head(h, x, hn, x0, mh, mx, w, bb):
    return pl.pallas_call(
        _head_body,
        grid=(NP // BN,),
        in_specs=[
            pl.BlockSpec((BN, HD), lambda i: (i, 0)),
            pl.BlockSpec((BN, 16), lambda i: (i, 0)),
            pl.BlockSpec((BN, INF), lambda i: (i, 0)),
            pl.BlockSpec((BN, 16), lambda i: (i, 0)),
            pl.BlockSpec((BN, INF), lambda i: (i, 0)),
            pl.BlockSpec((BN, 16), lambda i: (i, 0)),
            _full2((HD, INF)), _full2((1, INF)),
        ],
        out_specs=[
            pl.BlockSpec((BN, INF), lambda i: (i, 0)),
            pl.BlockSpec((BN, 16), lambda i: (i, 0)),
        ],
        out_shape=[
            jax.ShapeDtypeStruct((NP, INF), f32),
            jax.ShapeDtypeStruct((NP, 16), f32),
        ],
    )(h, x, hn, x0, mh, mx, w, bb)


def kernel(H_noisy, X_noisy, cond_embedding, edges, edge_types,
           generate_mask, batch_ids, beta, params):
    p = params
    padN = NP - NN
    hn = jnp.pad(H_noisy, ((0, padN), (0, 0)))
    cond = jnp.pad(cond_embedding, ((0, padN), (0, 0)))
    x0 = jnp.pad(X_noisy, ((0, padN), (0, 13)))
    betab = jnp.broadcast_to(jnp.pad(beta, (0, padN))[:, None], (NP, HD // 2))
    maskf = jnp.pad(generate_mask, (0, padN)).astype(f32)
    mh = jnp.broadcast_to(maskf[:, None], (NP, INF))
    mx = jnp.broadcast_to(maskf[:, None], (NP, 16))

    dstp = jnp.pad(edges[1], (0, EP - EDGES), constant_values=NN)
    srcp = jnp.pad(edges[0], (0, EP - EDGES), constant_values=NN)
    EPH = EP // 2
    dstgA = dstp[:EPH].reshape(EPH // GB, GB)
    dstgB = dstp[EPH:].reshape(EPH // GB, GB)
    srcgA = srcp[:EPH].reshape(EPH // GB, GB)
    srcgB = srcp[EPH:].reshape(EPH // GB, GB)
    dstsA = dstp[:EPH].reshape(EPH // SB, SB)
    dstsB = dstp[EPH:].reshape(EPH // SB, SB)
    tyf = jnp.broadcast_to(
        jnp.pad(edge_types, (0, EP - EDGES)).astype(f32)[:, None], (EP, 16))
    tyfA = tyf[:EP // 2]
    tyfB = tyf[EP // 2:]
    z128 = jnp.zeros((NP, 128), f32)

    w1 = p['inp_w1']
    h = _input_mlp(hn, cond, betab,
                   w1[0:INF], w1[INF:INF + HD],
                   w1[INF + HD:INF + HD + HD // 2], w1[INF + HD + HD // 2:],
                   p['inp_b1'].reshape(1, HD),
                   p['inp_w2'], p['inp_b2'].reshape(1, HD),
                   p['inp_w3'], p['inp_b3'].reshape(1, HD))
    x = x0

    for l in range(NLAYERS):
        ew1 = p[f'l{l}_ew1']
        wd = ew1[0:HD]
        ws = ew1[HD:2 * HD]
        wdist = ew1[2 * HD:2 * HD + 1].reshape(1, HD)
        we = ew1[2 * HD + 1:]
        xbp = jnp.pad(p[f'l{l}_xb'].reshape(1, 1), ((0, 0), (0, 127)))

        ad, as_ = _node_pre(h, x, wd, ws)
        ew_args = (p['edge_table'], we,
                   p[f'l{l}_eb1'].reshape(1, HD), wdist,
                   p[f'l{l}_ew2'], p[f'l{l}_eb2'].reshape(1, HD),
                   p[f'l{l}_xw'].reshape(1, HD), xbp)
        g1a = _sc_gather(ad, as_, dstgA, srcgA)
        g1b = _sc_gather(ad, as_, dstgB, srcgB)
        m5a = _edge_mlp(g1a, tyfA, *ew_args)
        m5b = _edge_mlp(g1b, tyfB, *ew_args)
        hma = _sc_scatter(m5a, dstsA, z128)
        hmb = _sc_scatter(m5b, dstsB, z128)
        hw1 = p[f'l{l}_hw1']
        h, x = _node_update(h, hma, hmb, x,
                            hw1[0:HD], hw1[HD:],
                            p[f'l{l}_hb1'].reshape(1, HD),
                            p[f'l{l}_hw2'], p[f'l{l}_hb2'].reshape(1, HD))

    eh, ex = _head(h, x, hn, x0, mh, mx,
                   p['h2i_w'], p['h2i_b'].reshape(1, INF))
    return eh[:NN], ex[:NN, :3]
